# baseline (speedup 1.0000x reference)
import functools

import jax
import jax.numpy as jnp
from jax import lax
from jax.experimental import pallas as pl
from jax.experimental.pallas import tpu as pltpu
from jax.experimental.pallas import tpu_sc as plsc

N_NODES = 100000
NNZ = 1600000
D = 128

NCHUNKS = 12
CHUNK = 8448
TILE_STRIPE = CHUNK // 16
LAST_ROWS = N_NODES - (NCHUNKS - 1) * CHUNK
LAST_STRIPE = 448
LAST_TAIL = LAST_ROWS - 15 * LAST_STRIPE
PER_TILE = NNZ // 16
SCAN_BLK = 2000
NBLK = PER_TILE // SCAN_BLK
NGRP = SCAN_BLK // 16
NSET = 4
STAGE = 96
FLUSH_AT = STAGE - 16


def _spmm_sc(rows, cols, vals, E):
    mesh = plsc.VectorSubcoreMesh(core_axis_name="c", subcore_axis_name="s")

    @functools.partial(
        pl.kernel,
        out_type=jax.ShapeDtypeStruct((N_NODES, D), jnp.float32),
        mesh=mesh,
        scratch_types=[
            pltpu.VMEM_SHARED((CHUNK, D), jnp.float32),
            pltpu.VMEM((SCAN_BLK,), jnp.int32),
            pltpu.VMEM((SCAN_BLK,), jnp.int32),
            pltpu.VMEM((SCAN_BLK,), jnp.float32),
            pltpu.VMEM((NSET, STAGE), jnp.int32),
            pltpu.VMEM((NSET, STAGE), jnp.int32),
            pltpu.VMEM((NSET, STAGE), jnp.float32),
            pltpu.VMEM((NSET, STAGE, D), jnp.float32),
            pltpu.SemaphoreType.DMA,
            pltpu.SemaphoreType.DMA,
        ],
        compiler_params=pltpu.CompilerParams(needs_layout_passes=False),
    )
    def spmm_kernel(rows_hbm, cols_hbm, vals_hbm, e_hbm, out_hbm,
                    acc, rows_v, cols_v, vals_v, cst, lst, vst, gbuf,
                    sem_g, sem_s):
        core = lax.axis_index("c")
        tid = lax.axis_index("s")
        z16i = jnp.zeros((16,), jnp.int32)
        z16f = jnp.zeros((16,), jnp.float32)

        def reset_stage(s):
            for g in range(STAGE // 16):
                cst[s, pl.ds(g * 16, 16)] = z16i
                lst[s, pl.ds(g * 16, 16)] = z16i
                vst[s, pl.ds(g * 16, 16)] = z16f

        for s0 in range(NSET):
            reset_stage(s0)

        def fire_gather(s):
            pltpu.async_copy(e_hbm.at[cst.at[s]], gbuf.at[s], sem_g)

        def process(s):
            pltpu.make_async_copy(e_hbm.at[cst.at[s]], gbuf.at[s],
                                  sem_g).wait()

            @plsc.parallel_loop(0, STAGE, 1, unroll=8)
            def _(r):
                bval = plsc.load_gather(vst, [z16i + s, z16i + r])
                for k in range(D // 16):
                    gbuf[s, r, pl.ds(k * 16, 16)] = (
                        gbuf[s, r, pl.ds(k * 16, 16)] * bval)

            pltpu.async_copy(gbuf.at[s], acc.at[lst.at[s]], sem_s, add=True)

        def wait_scatter(s):
            pltpu.make_async_copy(gbuf.at[s], acc.at[lst.at[s]],
                                  sem_s).wait()
            reset_stage(s)

        def chunk_body(c, carry):
            chunk_id = c * 2 + core
            lo = chunk_id * CHUNK
            hi = lo + CHUNK

            @plsc.parallel_loop(0, STAGE, 1, unroll=8)
            def _(r):
                for k in range(D // 16):
                    gbuf[0, r, pl.ds(k * 16, 16)] = z16f

            for z in range(TILE_STRIPE // STAGE):
                pltpu.sync_copy(
                    gbuf.at[0],
                    acc.at[pl.ds(tid * TILE_STRIPE + z * STAGE, STAGE)])
            pltpu.sync_copy(
                gbuf.at[0, pl.ds(0, TILE_STRIPE % STAGE)],
                acc.at[pl.ds(tid * TILE_STRIPE + (TILE_STRIPE // STAGE) * STAGE,
                             TILE_STRIPE % STAGE)])
            plsc.subcore_barrier()

            def flush_event(fc):
                fire_gather(fc & (NSET - 1))

                @pl.when(fc >= 2)
                def _():
                    process((fc - 2) & (NSET - 1))

                @pl.when(fc >= 3)
                def _():
                    wait_scatter((fc - 3) & (NSET - 1))

            def blk(b, st):
                off = tid * PER_TILE + b * SCAN_BLK
                pltpu.sync_copy(rows_hbm.at[pl.ds(off, SCAN_BLK)], rows_v)
                pltpu.sync_copy(cols_hbm.at[pl.ds(off, SCAN_BLK)], cols_v)
                pltpu.sync_copy(vals_hbm.at[pl.ds(off, SCAN_BLK)], vals_v)

                def grp(g, st):
                    nst, fc = st
                    cur = fc & (NSET - 1)
                    base = g * 16
                    r16 = rows_v[pl.ds(base, 16)]
                    m = (r16 >= lo) & (r16 < hi)
                    c16 = cols_v[pl.ds(base, 16)]
                    v16 = vals_v[pl.ds(base, 16)]
                    cnt = jnp.sum(m.astype(jnp.int32))
                    plsc.store_compressed(cst.at[cur, pl.ds(nst, 16)], c16,
                                          mask=m)
                    plsc.store_compressed(lst.at[cur, pl.ds(nst, 16)],
                                          r16 - lo, mask=m)
                    plsc.store_compressed(vst.at[cur, pl.ds(nst, 16)], v16,
                                          mask=m)
                    nst = nst + cnt
                    do_flush = nst > FLUSH_AT

                    @pl.when(do_flush)
                    def _():
                        flush_event(fc)

                    nst = jnp.where(do_flush, 0, nst).astype(jnp.int32)
                    fc = jnp.where(do_flush, fc + 1, fc).astype(jnp.int32)
                    return (nst, fc)

                return lax.fori_loop(0, NGRP, grp, st)

            nst, fc = lax.fori_loop(0, NBLK, blk,
                                    (jnp.int32(0), jnp.int32(0)))

            @pl.when(nst > 0)
            def _():
                flush_event(fc)

            fc = jnp.where(nst > 0, fc + 1, fc).astype(jnp.int32)

            @pl.when(fc >= 2)
            def _():
                process((fc - 2) & (NSET - 1))

            @pl.when(fc >= 1)
            def _():
                process((fc - 1) & (NSET - 1))

            @pl.when(fc >= 3)
            def _():
                wait_scatter((fc - 3) & (NSET - 1))

            @pl.when(fc >= 2)
            def _():
                wait_scatter((fc - 2) & (NSET - 1))

            @pl.when(fc >= 1)
            def _():
                wait_scatter((fc - 1) & (NSET - 1))

            plsc.subcore_barrier()

            is_last = chunk_id == NCHUNKS - 1

            @pl.when(jnp.logical_not(is_last))
            def _():
                pltpu.sync_copy(
                    acc.at[pl.ds(tid * TILE_STRIPE, TILE_STRIPE)],
                    out_hbm.at[pl.ds(lo + tid * TILE_STRIPE, TILE_STRIPE)])

            @pl.when(is_last & (tid < 15))
            def _():
                pltpu.sync_copy(
                    acc.at[pl.ds(tid * LAST_STRIPE, LAST_STRIPE)],
                    out_hbm.at[pl.ds(lo + tid * LAST_STRIPE, LAST_STRIPE)])

            @pl.when(is_last & (tid == 15))
            def _():
                pltpu.sync_copy(
                    acc.at[pl.ds(15 * LAST_STRIPE, LAST_TAIL)],
                    out_hbm.at[pl.ds(lo + 15 * LAST_STRIPE, LAST_TAIL)])

            plsc.subcore_barrier()
            return carry

        lax.fori_loop(0, NCHUNKS // 2, chunk_body, jnp.int32(0))

    return spmm_kernel(rows, cols, vals, E)


def _dense_tc(LE, E, Wc, W1, b):
    BLK = 2000

    def body(le_ref, e_ref, wc_ref, w1_ref, b_ref, o_ref):
        acc = lax.dot_general(le_ref[...], wc_ref[...],
                              (((1,), (1,)), ((), ())),
                              preferred_element_type=jnp.float32)
        acc = acc + lax.dot_general(e_ref[...], w1_ref[...],
                                    (((1,), (1,)), ((), ())),
                                    preferred_element_type=jnp.float32)
        o_ref[...] = acc + b_ref[...]

    return pl.pallas_call(
        body,
        grid=(N_NODES // BLK,),
        in_specs=[
            pl.BlockSpec((BLK, D), lambda i: (i, 0)),
            pl.BlockSpec((BLK, D), lambda i: (i, 0)),
            pl.BlockSpec((D, D), lambda i: (0, 0)),
            pl.BlockSpec((D, D), lambda i: (0, 0)),
            pl.BlockSpec((1, D), lambda i: (0, 0)),
        ],
        out_specs=pl.BlockSpec((BLK, D), lambda i: (i, 0)),
        out_shape=jax.ShapeDtypeStruct((N_NODES, D), jnp.float32),
    )(LE, E, Wc, W1, b)


def kernel(L_indices, L_values, E, W1, b1, W2, b2):
    rows = L_indices[0].astype(jnp.int32)
    cols = L_indices[1].astype(jnp.int32)
    LE = _spmm_sc(rows, cols, L_values.astype(jnp.float32), E)
    Wc = W1 + W2
    b = (b1 + b2).reshape(1, D)
    return _dense_tc(LE, E, Wc, W1, b)


# speedup vs baseline: 1.1715x; 1.1715x over previous
import functools

import jax
import jax.numpy as jnp
from jax import lax
from jax.experimental import pallas as pl
from jax.experimental.pallas import tpu as pltpu
from jax.experimental.pallas import tpu_sc as plsc

N_NODES = 100000
NNZ = 1600000
D = 128

NCHUNKS = 14
CHUNK = 7168
TILE_STRIPE = CHUNK // 16
LAST_ROWS = N_NODES - (NCHUNKS - 1) * CHUNK
LAST_STRIPE = 432
LAST_TAIL = LAST_ROWS - 15 * LAST_STRIPE
PER_TILE = NNZ // 16
SCAN_BLK = 2000
NBLK = PER_TILE // SCAN_BLK
NGRP = SCAN_BLK // 16
NROW = 4
SROW = 128
FLUSH_ROW_AT = SROW - 16


def _spmm_sc(rows, cols, vals, E):
    mesh = plsc.VectorSubcoreMesh(core_axis_name="c", subcore_axis_name="s")

    @functools.partial(
        pl.kernel,
        out_type=jax.ShapeDtypeStruct((N_NODES, D), jnp.float32),
        mesh=mesh,
        scratch_types=[
            pltpu.VMEM_SHARED((CHUNK, D), jnp.float32),
            pltpu.VMEM((SCAN_BLK,), jnp.int32),
            pltpu.VMEM((SCAN_BLK,), jnp.int32),
            pltpu.VMEM((SCAN_BLK,), jnp.float32),
            pltpu.VMEM((NROW * SROW,), jnp.int32),
            pltpu.VMEM((NROW, SROW), jnp.int32),
            pltpu.VMEM((NROW * SROW,), jnp.float32),
            pltpu.VMEM((NROW * SROW, D), jnp.float32),
            pltpu.SemaphoreType.DMA,
            pltpu.SemaphoreType.DMA,
        ],
        compiler_params=pltpu.CompilerParams(needs_layout_passes=False),
    )
    def spmm_kernel(rows_hbm, cols_hbm, vals_hbm, e_hbm, out_hbm,
                    acc, rows_v, cols_v, vals_v, cst, lst, vst, gbuf,
                    sem_g, sem_s):
        core = lax.axis_index("c")
        tid = lax.axis_index("s")
        z16i = jnp.zeros((16,), jnp.int32)
        z16f = jnp.zeros((16,), jnp.float32)

        def reset_stage():
            for g in range(NROW * SROW // 16):
                cst[pl.ds(g * 16, 16)] = z16i
                vst[pl.ds(g * 16, 16)] = z16f
            for s in range(NROW):
                for g in range(SROW // 16):
                    lst[s, pl.ds(g * 16, 16)] = z16i

        reset_stage()

        def flush():
            pltpu.async_copy(e_hbm.at[cst], gbuf, sem_g).wait()

            @plsc.parallel_loop(0, NROW * SROW, 1, unroll=8)
            def _(r):
                bval = plsc.load_gather(vst, [z16i + r])
                for k in range(D // 16):
                    gbuf[r, pl.ds(k * 16, 16)] = (
                        gbuf[r, pl.ds(k * 16, 16)] * bval)

            for j in range(NROW):
                pltpu.async_copy(gbuf.at[pl.ds(j * SROW, SROW)],
                                 acc.at[lst.at[j]], sem_s, add=True)
            for j in range(NROW):
                pltpu.make_async_copy(gbuf.at[pl.ds(j * SROW, SROW)],
                                      acc.at[lst.at[j]], sem_s).wait()
            reset_stage()

        def chunk_body(c, carry):
            chunk_id = c * 2 + core
            lo = chunk_id * CHUNK
            hi = lo + CHUNK

            @plsc.parallel_loop(0, SROW, 1, unroll=8)
            def _(r):
                for k in range(D // 16):
                    gbuf[r, pl.ds(k * 16, 16)] = z16f

            for z in range(TILE_STRIPE // SROW):
                pltpu.sync_copy(
                    gbuf.at[pl.ds(0, SROW)],
                    acc.at[pl.ds(tid * TILE_STRIPE + z * SROW, SROW)])
            pltpu.sync_copy(
                gbuf.at[pl.ds(0, TILE_STRIPE % SROW)],
                acc.at[pl.ds(tid * TILE_STRIPE + (TILE_STRIPE // SROW) * SROW,
                             TILE_STRIPE % SROW)])
            plsc.subcore_barrier()

            def blk(b, st):
                off = tid * PER_TILE + b * SCAN_BLK
                pltpu.sync_copy(rows_hbm.at[pl.ds(off, SCAN_BLK)], rows_v)
                pltpu.sync_copy(cols_hbm.at[pl.ds(off, SCAN_BLK)], cols_v)
                pltpu.sync_copy(vals_hbm.at[pl.ds(off, SCAN_BLK)], vals_v)

                def grp(g, st):
                    nst, jrow = st
                    base = g * 16
                    r16 = rows_v[pl.ds(base, 16)]
                    m = (r16 >= lo) & (r16 < hi)
                    c16 = cols_v[pl.ds(base, 16)]
                    v16 = vals_v[pl.ds(base, 16)]
                    cnt = jnp.sum(m.astype(jnp.int32))
                    pos = jrow * SROW + nst
                    plsc.store_compressed(cst.at[pl.ds(pos, 16)], c16,
                                          mask=m)
                    plsc.store_compressed(lst.at[jrow, pl.ds(nst, 16)],
                                          r16 - lo, mask=m)
                    plsc.store_compressed(vst.at[pl.ds(pos, 16)], v16,
                                          mask=m)
                    nst = nst + cnt
                    row_full = nst > FLUSH_ROW_AT
                    nst = jnp.where(row_full, 0, nst).astype(jnp.int32)
                    jrow = jnp.where(row_full, jrow + 1, jrow).astype(
                        jnp.int32)
                    do_flush = jrow >= NROW

                    @pl.when(do_flush)
                    def _():
                        flush()

                    jrow = jnp.where(do_flush, 0, jrow).astype(jnp.int32)
                    return (nst, jrow)

                return lax.fori_loop(0, NGRP, grp, st)

            nst, jrow = lax.fori_loop(0, NBLK, blk,
                                      (jnp.int32(0), jnp.int32(0)))

            @pl.when((nst > 0) | (jrow > 0))
            def _():
                flush()

            plsc.subcore_barrier()

            is_last = chunk_id == NCHUNKS - 1

            @pl.when(jnp.logical_not(is_last))
            def _():
                pltpu.sync_copy(
                    acc.at[pl.ds(tid * TILE_STRIPE, TILE_STRIPE)],
                    out_hbm.at[pl.ds(lo + tid * TILE_STRIPE, TILE_STRIPE)])

            @pl.when(is_last & (tid < 15))
            def _():
                pltpu.sync_copy(
                    acc.at[pl.ds(tid * LAST_STRIPE, LAST_STRIPE)],
                    out_hbm.at[pl.ds(lo + tid * LAST_STRIPE, LAST_STRIPE)])

            @pl.when(is_last & (tid == 15))
            def _():
                pltpu.sync_copy(
                    acc.at[pl.ds(15 * LAST_STRIPE, LAST_TAIL)],
                    out_hbm.at[pl.ds(lo + 15 * LAST_STRIPE, LAST_TAIL)])

            plsc.subcore_barrier()
            return carry

        lax.fori_loop(0, NCHUNKS // 2, chunk_body, jnp.int32(0))

    return spmm_kernel(rows, cols, vals, E)


def _dense_tc(LE, E, Wc, W1, b):
    BLK = 2000

    def body(le_ref, e_ref, wc_ref, w1_ref, b_ref, o_ref):
        acc = lax.dot_general(le_ref[...], wc_ref[...],
                              (((1,), (1,)), ((), ())),
                              preferred_element_type=jnp.float32)
        acc = acc + lax.dot_general(e_ref[...], w1_ref[...],
                                    (((1,), (1,)), ((), ())),
                                    preferred_element_type=jnp.float32)
        o_ref[...] = acc + b_ref[...]

    return pl.pallas_call(
        body,
        grid=(N_NODES // BLK,),
        in_specs=[
            pl.BlockSpec((BLK, D), lambda i: (i, 0)),
            pl.BlockSpec((BLK, D), lambda i: (i, 0)),
            pl.BlockSpec((D, D), lambda i: (0, 0)),
            pl.BlockSpec((D, D), lambda i: (0, 0)),
            pl.BlockSpec((1, D), lambda i: (0, 0)),
        ],
        out_specs=pl.BlockSpec((BLK, D), lambda i: (i, 0)),
        out_shape=jax.ShapeDtypeStruct((N_NODES, D), jnp.float32),
    )(LE, E, Wc, W1, b)


def kernel(L_indices, L_values, E, W1, b1, W2, b2):
    rows = L_indices[0].astype(jnp.int32)
    cols = L_indices[1].astype(jnp.int32)
    LE = _spmm_sc(rows, cols, L_values.astype(jnp.float32), E)
    Wc = W1 + W2
    b = (b1 + b2).reshape(1, D)
    return _dense_tc(LE, E, Wc, W1, b)


# speedup vs baseline: 1.3543x; 1.1561x over previous
import functools

import jax
import jax.numpy as jnp
from jax import lax
from jax.experimental import pallas as pl
from jax.experimental.pallas import tpu as pltpu
from jax.experimental.pallas import tpu_sc as plsc

N_NODES = 100000
NNZ = 1600000
D = 128

NCHUNKS = 10
CHUNK = 10240
TILE_STRIPE = CHUNK // 16
LAST_ROWS = N_NODES - (NCHUNKS - 1) * CHUNK
LAST_STRIPE = 496
LAST_TAIL = LAST_ROWS - 15 * LAST_STRIPE
PER_TILE = NNZ // 16
SCAN_BLK = 2000
NBLK = PER_TILE // SCAN_BLK
NGRP = SCAN_BLK // 16
STAGE = 128
FLUSH_AT = 112


def _spmm_sc(rows, cols, vals, E):
    mesh = plsc.VectorSubcoreMesh(core_axis_name="c", subcore_axis_name="s")

    @functools.partial(
        pl.kernel,
        out_type=jax.ShapeDtypeStruct((N_NODES, D), jnp.float32),
        mesh=mesh,
        scratch_types=[
            pltpu.VMEM_SHARED((CHUNK, D), jnp.float32),
            pltpu.VMEM((SCAN_BLK,), jnp.int32),
            pltpu.VMEM((SCAN_BLK,), jnp.int32),
            pltpu.VMEM((SCAN_BLK,), jnp.float32),
            pltpu.VMEM((2, STAGE), jnp.int32),
            pltpu.VMEM((2, STAGE), jnp.int32),
            pltpu.VMEM((2, STAGE), jnp.float32),
            pltpu.VMEM((2, STAGE, D), jnp.float32),
            pltpu.SemaphoreType.DMA,
        ],
        compiler_params=pltpu.CompilerParams(needs_layout_passes=False),
    )
    def spmm_kernel(rows_hbm, cols_hbm, vals_hbm, e_hbm, out_hbm,
                    acc, rows_v, cols_v, vals_v, cst, lst, vst, gbuf,
                    sem):
        core = lax.axis_index("c")
        tid = lax.axis_index("s")
        z16i = jnp.zeros((16,), jnp.int32)
        z16f = jnp.zeros((16,), jnp.float32)

        def zero_gbuf0():
            def zrow(i, carry):
                for k in range(D // 16):
                    gbuf[0, i, pl.ds(k * 16, 16)] = z16f
                return carry

            lax.fori_loop(0, STAGE, zrow, jnp.int32(0))

        def reset_stage(s):
            for g in range(STAGE // 16):
                cst[s, pl.ds(g * 16, 16)] = z16i
                lst[s, pl.ds(g * 16, 16)] = z16i
                vst[s, pl.ds(g * 16, 16)] = z16f

        reset_stage(0)
        reset_stage(1)

        def fire_gather(s):
            pltpu.async_copy(e_hbm.at[cst.at[s]], gbuf.at[s], sem)

        def drain_process(s):
            pltpu.make_async_copy(e_hbm.at[cst.at[s]], gbuf.at[s], sem).wait()

            @plsc.parallel_loop(0, STAGE, 1, unroll=8)
            def _(r):
                bval = plsc.load_gather(vst, [z16i + s, z16i + r])
                for k in range(D // 16):
                    gbuf[s, r, pl.ds(k * 16, 16)] = (
                        gbuf[s, r, pl.ds(k * 16, 16)] * bval)

            pltpu.sync_copy(gbuf.at[s], acc.at[lst.at[s]], add=True)
            reset_stage(s)

        def chunk_body(c, carry):
            chunk_id = c * 2 + core
            lo = chunk_id * CHUNK
            hi = lo + CHUNK

            zero_gbuf0()
            for z in range(TILE_STRIPE // STAGE):
                pltpu.sync_copy(
                    gbuf.at[0],
                    acc.at[pl.ds(tid * TILE_STRIPE + z * STAGE, STAGE)])
            plsc.subcore_barrier()

            def blk(b, st):
                off = tid * PER_TILE + b * SCAN_BLK
                pltpu.sync_copy(rows_hbm.at[pl.ds(off, SCAN_BLK)], rows_v)
                pltpu.sync_copy(cols_hbm.at[pl.ds(off, SCAN_BLK)], cols_v)
                pltpu.sync_copy(vals_hbm.at[pl.ds(off, SCAN_BLK)], vals_v)

                def grp(g, st):
                    nst, cur, infl = st
                    base = g * 16
                    r16 = rows_v[pl.ds(base, 16)]
                    m = (r16 >= lo) & (r16 < hi)
                    c16 = cols_v[pl.ds(base, 16)]
                    v16 = vals_v[pl.ds(base, 16)]
                    cnt = jnp.sum(m.astype(jnp.int32))
                    plsc.store_compressed(cst.at[cur, pl.ds(nst, 16)], c16,
                                          mask=m)
                    plsc.store_compressed(lst.at[cur, pl.ds(nst, 16)],
                                          r16 - lo, mask=m)
                    plsc.store_compressed(vst.at[cur, pl.ds(nst, 16)], v16,
                                          mask=m)
                    nst = nst + cnt
                    do_flush = nst > FLUSH_AT

                    @pl.when(do_flush)
                    def _():
                        @pl.when(infl == 1)
                        def _():
                            drain_process(1 - cur)

                        fire_gather(cur)

                    nst = jnp.where(do_flush, 0, nst).astype(jnp.int32)
                    cur2 = jnp.where(do_flush, 1 - cur, cur).astype(jnp.int32)
                    infl2 = jnp.where(do_flush, 1, infl).astype(jnp.int32)
                    return (nst, cur2, infl2)

                return lax.fori_loop(0, NGRP, grp, st)

            nst, cur, infl = lax.fori_loop(
                0, NBLK, blk, (jnp.int32(0), jnp.int32(0), jnp.int32(0)))

            @pl.when(infl == 1)
            def _():
                drain_process(1 - cur)

            @pl.when(nst > 0)
            def _():
                fire_gather(cur)
                drain_process(cur)

            plsc.subcore_barrier()

            is_last = chunk_id == NCHUNKS - 1

            @pl.when(jnp.logical_not(is_last))
            def _():
                pltpu.sync_copy(
                    acc.at[pl.ds(tid * TILE_STRIPE, TILE_STRIPE)],
                    out_hbm.at[pl.ds(lo + tid * TILE_STRIPE, TILE_STRIPE)])

            @pl.when(is_last & (tid < 15))
            def _():
                pltpu.sync_copy(
                    acc.at[pl.ds(tid * LAST_STRIPE, LAST_STRIPE)],
                    out_hbm.at[pl.ds(lo + tid * LAST_STRIPE, LAST_STRIPE)])

            @pl.when(is_last & (tid == 15))
            def _():
                pltpu.sync_copy(
                    acc.at[pl.ds(15 * LAST_STRIPE, LAST_TAIL)],
                    out_hbm.at[pl.ds(lo + 15 * LAST_STRIPE, LAST_TAIL)])

            plsc.subcore_barrier()
            return carry

        lax.fori_loop(0, NCHUNKS // 2, chunk_body, jnp.int32(0))

    return spmm_kernel(rows, cols, vals, E)


def _dense_tc(LE, E, Wc, W1, b):
    BLK = 2000

    def body(le_ref, e_ref, wc_ref, w1_ref, b_ref, o_ref):
        acc = lax.dot_general(le_ref[...], wc_ref[...],
                              (((1,), (1,)), ((), ())),
                              preferred_element_type=jnp.float32)
        acc = acc + lax.dot_general(e_ref[...], w1_ref[...],
                                    (((1,), (1,)), ((), ())),
                                    preferred_element_type=jnp.float32)
        o_ref[...] = acc + b_ref[...]

    return pl.pallas_call(
        body,
        grid=(N_NODES // BLK,),
        in_specs=[
            pl.BlockSpec((BLK, D), lambda i: (i, 0)),
            pl.BlockSpec((BLK, D), lambda i: (i, 0)),
            pl.BlockSpec((D, D), lambda i: (0, 0)),
            pl.BlockSpec((D, D), lambda i: (0, 0)),
            pl.BlockSpec((1, D), lambda i: (0, 0)),
        ],
        out_specs=pl.BlockSpec((BLK, D), lambda i: (i, 0)),
        out_shape=jax.ShapeDtypeStruct((N_NODES, D), jnp.float32),
    )(LE, E, Wc, W1, b)


def kernel(L_indices, L_values, E, W1, b1, W2, b2):
    rows = L_indices[0].astype(jnp.int32)
    cols = L_indices[1].astype(jnp.int32)
    LE = _spmm_sc(rows, cols, L_values.astype(jnp.float32), E)
    Wc = W1 + W2
    b = (b1 + b2).reshape(1, D)
    return _dense_tc(LE, E, Wc, W1, b)


# speedup vs baseline: 1.3839x; 1.0219x over previous
import functools

import jax
import jax.numpy as jnp
from jax import lax
from jax.experimental import pallas as pl
from jax.experimental.pallas import tpu as pltpu
from jax.experimental.pallas import tpu_sc as plsc

N_NODES = 100000
NNZ = 1600000
D = 128

NCHUNKS = 8
CHUNK = 12544
TILE_STRIPE = CHUNK // 16
LAST_ROWS = N_NODES - (NCHUNKS - 1) * CHUNK
LAST_STRIPE = 768
LAST_TAIL = LAST_ROWS - 15 * LAST_STRIPE
ZROWS = 112
PER_TILE = NNZ // 16
SCAN_BLK = 2000
NBLK = PER_TILE // SCAN_BLK
NGRP = SCAN_BLK // 16
STAGE = 128
FLUSH_AT = 112


def _spmm_sc(rows, cols, vals, E):
    mesh = plsc.VectorSubcoreMesh(core_axis_name="c", subcore_axis_name="s")

    @functools.partial(
        pl.kernel,
        out_type=jax.ShapeDtypeStruct((N_NODES, D), jnp.float32),
        mesh=mesh,
        scratch_types=[
            pltpu.VMEM_SHARED((CHUNK, D), jnp.float32),
            pltpu.VMEM((SCAN_BLK,), jnp.int32),
            pltpu.VMEM((SCAN_BLK,), jnp.int32),
            pltpu.VMEM((SCAN_BLK,), jnp.float32),
            pltpu.VMEM((SCAN_BLK,), jnp.int32),
            pltpu.VMEM((SCAN_BLK,), jnp.int32),
            pltpu.VMEM((SCAN_BLK,), jnp.float32),
            pltpu.VMEM((STAGE,), jnp.int32),
            pltpu.VMEM((STAGE,), jnp.int32),
            pltpu.VMEM((STAGE,), jnp.float32),
            pltpu.VMEM((STAGE, D), jnp.float32),
            pltpu.SemaphoreType.DMA,
            pltpu.SemaphoreType.DMA,
        ],
        compiler_params=pltpu.CompilerParams(needs_layout_passes=False),
    )
    def spmm_kernel(rows_hbm, cols_hbm, vals_hbm, e_hbm, out_hbm,
                    acc, rows_v0, cols_v0, vals_v0, rows_v1, cols_v1,
                    vals_v1, cst, lst, vst, gbuf, sem, sem_e):
        core = lax.axis_index("c")
        tid = lax.axis_index("s")
        z16i = jnp.zeros((16,), jnp.int32)
        z16f = jnp.zeros((16,), jnp.float32)

        def zero_gbuf():
            def zrow(i, carry):
                for k in range(D // 16):
                    gbuf[i, pl.ds(k * 16, 16)] = z16f
                return carry

            lax.fori_loop(0, STAGE, zrow, jnp.int32(0))

        def reset_stage():
            for g in range(STAGE // 16):
                cst[pl.ds(g * 16, 16)] = z16i
                lst[pl.ds(g * 16, 16)] = z16i
                vst[pl.ds(g * 16, 16)] = z16f

        reset_stage()

        def flush():
            pltpu.async_copy(e_hbm.at[cst], gbuf, sem).wait()

            def scale(r, carry):
                bval = plsc.load_gather(vst, [z16i + r])
                for k in range(D // 16):
                    gbuf[r, pl.ds(k * 16, 16)] = (
                        gbuf[r, pl.ds(k * 16, 16)] * bval)
                return carry

            lax.fori_loop(0, STAGE, scale, jnp.int32(0))
            pltpu.sync_copy(gbuf, acc.at[lst], add=True)
            reset_stage()

        def chunk_body(c, carry):
            chunk_id = c * 2 + core
            lo = chunk_id * CHUNK
            hi = lo + CHUNK

            zero_gbuf()
            for z in range(TILE_STRIPE // ZROWS):
                pltpu.sync_copy(
                    gbuf.at[pl.ds(0, ZROWS)],
                    acc.at[pl.ds(tid * TILE_STRIPE + z * ZROWS, ZROWS)])
            plsc.subcore_barrier()

            def fire_edge_load(b):
                off = tid * PER_TILE + b * SCAN_BLK

                @pl.when(b & 1 == 0)
                def _():
                    pltpu.async_copy(rows_hbm.at[pl.ds(off, SCAN_BLK)],
                                     rows_v0, sem_e)
                    pltpu.async_copy(cols_hbm.at[pl.ds(off, SCAN_BLK)],
                                     cols_v0, sem_e)
                    pltpu.async_copy(vals_hbm.at[pl.ds(off, SCAN_BLK)],
                                     vals_v0, sem_e)

                @pl.when(b & 1 == 1)
                def _():
                    pltpu.async_copy(rows_hbm.at[pl.ds(off, SCAN_BLK)],
                                     rows_v1, sem_e)
                    pltpu.async_copy(cols_hbm.at[pl.ds(off, SCAN_BLK)],
                                     cols_v1, sem_e)
                    pltpu.async_copy(vals_hbm.at[pl.ds(off, SCAN_BLK)],
                                     vals_v1, sem_e)

            def wait_edge_load(b):
                off = tid * PER_TILE + b * SCAN_BLK
                pltpu.make_async_copy(rows_hbm.at[pl.ds(off, SCAN_BLK)],
                                      rows_v0, sem_e).wait()
                pltpu.make_async_copy(cols_hbm.at[pl.ds(off, SCAN_BLK)],
                                      cols_v0, sem_e).wait()
                pltpu.make_async_copy(vals_hbm.at[pl.ds(off, SCAN_BLK)],
                                      vals_v0, sem_e).wait()

            fire_edge_load(jnp.int32(0))

            def blk(b, nst):
                wait_edge_load(b)

                @pl.when(b < NBLK - 1)
                def _():
                    fire_edge_load(b + 1)

                def scan_groups(rows_v, cols_v, vals_v, nst0):
                    def grp(g, nst):
                        base = g * 16
                        r16 = rows_v[pl.ds(base, 16)]
                        m = (r16 >= lo) & (r16 < hi)
                        c16 = cols_v[pl.ds(base, 16)]
                        v16 = vals_v[pl.ds(base, 16)]
                        cnt = jnp.sum(m.astype(jnp.int32))
                        plsc.store_compressed(cst.at[pl.ds(nst, 16)], c16, mask=m)
                        plsc.store_compressed(lst.at[pl.ds(nst, 16)], r16 - lo,
                                          mask=m)
                        plsc.store_compressed(vst.at[pl.ds(nst, 16)], v16, mask=m)
                        nst = nst + cnt
                        do_flush = nst > FLUSH_AT

                        @pl.when(do_flush)
                        def _():
                            flush()

                        return jnp.where(do_flush, 0, nst).astype(jnp.int32)

                    return lax.fori_loop(0, NGRP, grp, nst0)

                return lax.cond(
                    b & 1 == 0,
                    lambda n: scan_groups(rows_v0, cols_v0, vals_v0, n),
                    lambda n: scan_groups(rows_v1, cols_v1, vals_v1, n),
                    nst)

            nst = lax.fori_loop(0, NBLK, blk, jnp.int32(0))

            @pl.when(nst > 0)
            def _():
                flush()

            plsc.subcore_barrier()

            is_last = chunk_id == NCHUNKS - 1

            @pl.when(jnp.logical_not(is_last))
            def _():
                pltpu.sync_copy(
                    acc.at[pl.ds(tid * TILE_STRIPE, TILE_STRIPE)],
                    out_hbm.at[pl.ds(lo + tid * TILE_STRIPE, TILE_STRIPE)])

            @pl.when(is_last & (tid < 15))
            def _():
                pltpu.sync_copy(
                    acc.at[pl.ds(tid * LAST_STRIPE, LAST_STRIPE)],
                    out_hbm.at[pl.ds(lo + tid * LAST_STRIPE, LAST_STRIPE)])

            @pl.when(is_last & (tid == 15))
            def _():
                pltpu.sync_copy(
                    acc.at[pl.ds(15 * LAST_STRIPE, LAST_TAIL)],
                    out_hbm.at[pl.ds(lo + 15 * LAST_STRIPE, LAST_TAIL)])

            plsc.subcore_barrier()
            return carry

        lax.fori_loop(0, NCHUNKS // 2, chunk_body, jnp.int32(0))

    return spmm_kernel(rows, cols, vals, E)


def _dense_tc(LE, E, Wc, W1, b):
    BLK = 2000

    def body(le_ref, e_ref, wc_ref, w1_ref, b_ref, o_ref):
        acc = lax.dot_general(le_ref[...], wc_ref[...],
                              (((1,), (1,)), ((), ())),
                              preferred_element_type=jnp.float32)
        acc = acc + lax.dot_general(e_ref[...], w1_ref[...],
                                    (((1,), (1,)), ((), ())),
                                    preferred_element_type=jnp.float32)
        o_ref[...] = acc + b_ref[...]

    return pl.pallas_call(
        body,
        grid=(N_NODES // BLK,),
        in_specs=[
            pl.BlockSpec((BLK, D), lambda i: (i, 0)),
            pl.BlockSpec((BLK, D), lambda i: (i, 0)),
            pl.BlockSpec((D, D), lambda i: (0, 0)),
            pl.BlockSpec((D, D), lambda i: (0, 0)),
            pl.BlockSpec((1, D), lambda i: (0, 0)),
        ],
        out_specs=pl.BlockSpec((BLK, D), lambda i: (i, 0)),
        out_shape=jax.ShapeDtypeStruct((N_NODES, D), jnp.float32),
    )(LE, E, Wc, W1, b)


def kernel(L_indices, L_values, E, W1, b1, W2, b2):
    rows = L_indices[0].astype(jnp.int32)
    cols = L_indices[1].astype(jnp.int32)
    LE = _spmm_sc(rows, cols, L_values.astype(jnp.float32), E)
    Wc = W1 + W2
    b = (b1 + b2).reshape(1, D)
    return _dense_tc(LE, E, Wc, W1, b)


# speedup vs baseline: 1.3875x; 1.0026x over previous
import functools

import jax
import jax.numpy as jnp
from jax import lax
from jax.experimental import pallas as pl
from jax.experimental.pallas import tpu as pltpu
from jax.experimental.pallas import tpu_sc as plsc

N_NODES = 100000
NNZ = 1600000
D = 128

NCHUNKS = 8
CHUNK = 12544
TILE_STRIPE = CHUNK // 16
LAST_ROWS = N_NODES - (NCHUNKS - 1) * CHUNK
LAST_STRIPE = 768
LAST_TAIL = LAST_ROWS - 15 * LAST_STRIPE
ZROWS = 112
PER_TILE = NNZ // 16
SCAN_BLK = 2000
NBLK = PER_TILE // SCAN_BLK
NGRP = SCAN_BLK // 16
STAGE = 128
FLUSH_AT = 112


def _spmm_sc(rows, cols, vals, E):
    mesh = plsc.VectorSubcoreMesh(core_axis_name="c", subcore_axis_name="s")

    @functools.partial(
        pl.kernel,
        out_type=jax.ShapeDtypeStruct((N_NODES, D), jnp.float32),
        mesh=mesh,
        scratch_types=[
            pltpu.VMEM_SHARED((CHUNK, D), jnp.float32),
            pltpu.VMEM((SCAN_BLK,), jnp.int32),
            pltpu.VMEM((SCAN_BLK,), jnp.int32),
            pltpu.VMEM((SCAN_BLK,), jnp.float32),
            pltpu.VMEM((STAGE,), jnp.int32),
            pltpu.VMEM((STAGE,), jnp.int32),
            pltpu.VMEM((STAGE,), jnp.float32),
            pltpu.VMEM((STAGE, D), jnp.float32),
            pltpu.SemaphoreType.DMA,
        ],
        compiler_params=pltpu.CompilerParams(needs_layout_passes=False),
    )
    def spmm_kernel(rows_hbm, cols_hbm, vals_hbm, e_hbm, out_hbm,
                    acc, rows_v, cols_v, vals_v, cst, lst, vst, gbuf,
                    sem):
        core = lax.axis_index("c")
        tid = lax.axis_index("s")
        z16i = jnp.zeros((16,), jnp.int32)
        z16f = jnp.zeros((16,), jnp.float32)

        def zero_gbuf():
            def zrow(i, carry):
                for k in range(D // 16):
                    gbuf[i, pl.ds(k * 16, 16)] = z16f
                return carry

            lax.fori_loop(0, STAGE, zrow, jnp.int32(0))

        def reset_stage():
            for g in range(STAGE // 16):
                cst[pl.ds(g * 16, 16)] = z16i
                lst[pl.ds(g * 16, 16)] = z16i
                vst[pl.ds(g * 16, 16)] = z16f

        reset_stage()

        def flush():
            pltpu.async_copy(e_hbm.at[cst], gbuf, sem).wait()

            def scale(r, carry):
                bval = plsc.load_gather(vst, [z16i + r])
                for k in range(D // 16):
                    gbuf[r, pl.ds(k * 16, 16)] = (
                        gbuf[r, pl.ds(k * 16, 16)] * bval)
                return carry

            lax.fori_loop(0, STAGE, scale, jnp.int32(0))
            pltpu.sync_copy(gbuf, acc.at[lst], add=True)
            reset_stage()

        def chunk_body(c, carry):
            chunk_id = c * 2 + core
            lo = chunk_id * CHUNK
            hi = lo + CHUNK

            zero_gbuf()
            for z in range(TILE_STRIPE // ZROWS):
                pltpu.sync_copy(
                    gbuf.at[pl.ds(0, ZROWS)],
                    acc.at[pl.ds(tid * TILE_STRIPE + z * ZROWS, ZROWS)])
            plsc.subcore_barrier()

            def blk(b, nst):
                off = tid * PER_TILE + b * SCAN_BLK
                pltpu.sync_copy(rows_hbm.at[pl.ds(off, SCAN_BLK)], rows_v)
                pltpu.sync_copy(cols_hbm.at[pl.ds(off, SCAN_BLK)], cols_v)
                pltpu.sync_copy(vals_hbm.at[pl.ds(off, SCAN_BLK)], vals_v)

                def grp(g, nst):
                    base = g * 16
                    r16 = rows_v[pl.ds(base, 16)]
                    m = (r16 >= lo) & (r16 < hi)
                    c16 = cols_v[pl.ds(base, 16)]
                    v16 = vals_v[pl.ds(base, 16)]
                    cnt = jnp.sum(m.astype(jnp.int32))
                    plsc.store_compressed(cst.at[pl.ds(nst, 16)], c16, mask=m)
                    plsc.store_compressed(lst.at[pl.ds(nst, 16)], r16 - lo,
                                          mask=m)
                    plsc.store_compressed(vst.at[pl.ds(nst, 16)], v16, mask=m)
                    nst = nst + cnt
                    do_flush = nst > FLUSH_AT

                    @pl.when(do_flush)
                    def _():
                        flush()

                    return jnp.where(do_flush, 0, nst).astype(jnp.int32)

                return lax.fori_loop(0, NGRP, grp, nst)

            nst = lax.fori_loop(0, NBLK, blk, jnp.int32(0))

            @pl.when(nst > 0)
            def _():
                flush()

            plsc.subcore_barrier()

            is_last = chunk_id == NCHUNKS - 1

            @pl.when(jnp.logical_not(is_last))
            def _():
                pltpu.sync_copy(
                    acc.at[pl.ds(tid * TILE_STRIPE, TILE_STRIPE)],
                    out_hbm.at[pl.ds(lo + tid * TILE_STRIPE, TILE_STRIPE)])

            @pl.when(is_last & (tid < 15))
            def _():
                pltpu.sync_copy(
                    acc.at[pl.ds(tid * LAST_STRIPE, LAST_STRIPE)],
                    out_hbm.at[pl.ds(lo + tid * LAST_STRIPE, LAST_STRIPE)])

            @pl.when(is_last & (tid == 15))
            def _():
                pltpu.sync_copy(
                    acc.at[pl.ds(15 * LAST_STRIPE, LAST_TAIL)],
                    out_hbm.at[pl.ds(lo + 15 * LAST_STRIPE, LAST_TAIL)])

            plsc.subcore_barrier()
            return carry

        lax.fori_loop(0, NCHUNKS // 2, chunk_body, jnp.int32(0))

    return spmm_kernel(rows, cols, vals, E)


def _dense_tc(LE, E, Wc, W1, b):
    BLK = 2000

    def body(le_ref, e_ref, wc_ref, w1_ref, b_ref, o_ref):
        acc = lax.dot_general(le_ref[...], wc_ref[...],
                              (((1,), (1,)), ((), ())),
                              preferred_element_type=jnp.float32)
        acc = acc + lax.dot_general(e_ref[...], w1_ref[...],
                                    (((1,), (1,)), ((), ())),
                                    preferred_element_type=jnp.float32)
        o_ref[...] = acc + b_ref[...]

    return pl.pallas_call(
        body,
        grid=(N_NODES // BLK,),
        in_specs=[
            pl.BlockSpec((BLK, D), lambda i: (i, 0)),
            pl.BlockSpec((BLK, D), lambda i: (i, 0)),
            pl.BlockSpec((D, D), lambda i: (0, 0)),
            pl.BlockSpec((D, D), lambda i: (0, 0)),
            pl.BlockSpec((1, D), lambda i: (0, 0)),
        ],
        out_specs=pl.BlockSpec((BLK, D), lambda i: (i, 0)),
        out_shape=jax.ShapeDtypeStruct((N_NODES, D), jnp.float32),
    )(LE, E, Wc, W1, b)


def kernel(L_indices, L_values, E, W1, b1, W2, b2):
    rows = L_indices[0].astype(jnp.int32)
    cols = L_indices[1].astype(jnp.int32)
    LE = _spmm_sc(rows, cols, L_values.astype(jnp.float32), E)
    Wc = W1 + W2
    b = (b1 + b2).reshape(1, D)
    return _dense_tc(LE, E, Wc, W1, b)
